# Initial kernel scaffold; baseline (speedup 1.0000x reference)
import jax
import jax.numpy as jnp
from jax.experimental import pallas as pl


def kernel(x, edge_index, batch, Wl0, Wr0, b0, Wl1, Wr1, b1, Wl2, Wr2, b2, gamma, beta, fcW, fcb):
    raise NotImplementedError("write your pallas kernel here")


# speedup vs baseline: 3.4854x; 3.4854x over previous
import functools

import jax
import jax.numpy as jnp
from jax import lax
from jax.experimental import pallas as pl
from jax.experimental.pallas import tpu as pltpu
from jax.experimental.pallas import tpu_sc as plsc

N = 10000
E = 320000
D = 128
H = 256
G = 64

NUM_TILES = 16
C = 80
NPAD = 10240
RPT = NPAD // NUM_TILES

_mesh = plsc.VectorSubcoreMesh(core_axis_name="c", subcore_axis_name="s")


@functools.partial(
    pl.kernel,
    mesh=_mesh,
    out_type=jax.ShapeDtypeStruct((2 * NPAD, 128), jnp.float32),
    scratch_types=[
        pltpu.VMEM((C,), jnp.int32),
        pltpu.VMEM((C, 128), jnp.float32),
        pltpu.VMEM_SHARED((NPAD, 128), jnp.float32),
    ],
)
def _sc_count(dst_hbm, ones_hbm, zeros_hbm, out_hbm, idx_v, ones_v, acc_sh):
    cid = lax.axis_index("c")
    sid = lax.axis_index("s")
    ept = E // 32
    chunks = ept // C

    pltpu.sync_copy(zeros_hbm.at[pl.ds(sid * RPT, RPT)],
                    acc_sh.at[pl.ds(sid * RPT, RPT)])
    pltpu.sync_copy(ones_hbm, ones_v)
    plsc.subcore_barrier()

    def chunk(k, carry):
        base = cid * (E // 2) + sid * ept + k * C
        pltpu.sync_copy(dst_hbm.at[pl.ds(base, C)], idx_v)
        pltpu.sync_copy(ones_v, acc_sh.at[idx_v], add=True)
        return carry

    lax.fori_loop(0, chunks, chunk, 0)
    plsc.subcore_barrier()
    pltpu.sync_copy(acc_sh.at[pl.ds(sid * RPT, RPT)],
                    out_hbm.at[pl.ds(cid * NPAD + sid * RPT, RPT)])


@functools.partial(
    pl.kernel,
    mesh=_mesh,
    out_type=jax.ShapeDtypeStruct((2 * NPAD, D), jnp.float32),
    scratch_types=[
        pltpu.VMEM((C,), jnp.int32),
        pltpu.VMEM((C,), jnp.int32),
        pltpu.VMEM((C, D), jnp.float32),
        pltpu.VMEM_SHARED((NPAD, D), jnp.float32),
        pltpu.SemaphoreType.DMA,
    ],
)
def _sc_agg_l0(x_hbm, src_hbm, dst_hbm, zeros_hbm, out_hbm,
               src_v, dst_v, rows_v, acc_sh, sem):
    cid = lax.axis_index("c")
    sid = lax.axis_index("s")
    ept = E // 32
    chunks = ept // C

    pltpu.sync_copy(zeros_hbm.at[pl.ds(sid * RPT, RPT)],
                    acc_sh.at[pl.ds(sid * RPT, RPT)])
    plsc.subcore_barrier()

    def chunk(k, carry):
        base = cid * (E // 2) + sid * ept + k * C
        pltpu.sync_copy(src_hbm.at[pl.ds(base, C)], src_v)
        pltpu.sync_copy(dst_hbm.at[pl.ds(base, C)], dst_v)
        pltpu.async_copy(x_hbm.at[src_v], rows_v, sem).wait()
        pltpu.sync_copy(rows_v, acc_sh.at[dst_v], add=True)
        return carry

    lax.fori_loop(0, chunks, chunk, 0)
    plsc.subcore_barrier()
    pltpu.sync_copy(acc_sh.at[pl.ds(sid * RPT, RPT)],
                    out_hbm.at[pl.ds(cid * NPAD + sid * RPT, RPT)])


@functools.partial(
    pl.kernel,
    mesh=_mesh,
    out_type=jax.ShapeDtypeStruct((2 * NPAD, H // 2), jnp.float32),
    scratch_types=[
        pltpu.VMEM((C,), jnp.int32),
        pltpu.VMEM((C,), jnp.int32),
        pltpu.VMEM((C, H // 2), jnp.float32),
        pltpu.VMEM_SHARED((NPAD, H // 2), jnp.float32),
        pltpu.SemaphoreType.DMA,
    ],
)
def _sc_agg(h_hbm, srcs_hbm, dst_hbm, zeros_hbm, out_hbm,
            src_v, dst_v, rows_v, acc_sh, sem):
    cid = lax.axis_index("c")
    sid = lax.axis_index("s")
    ept = E // NUM_TILES
    chunks = ept // C

    pltpu.sync_copy(zeros_hbm.at[pl.ds(sid * RPT, RPT)],
                    acc_sh.at[pl.ds(sid * RPT, RPT)])
    plsc.subcore_barrier()

    def chunk(k, carry):
        base = sid * ept + k * C
        pltpu.sync_copy(srcs_hbm.at[pl.ds(cid * E + base, C)], src_v)
        pltpu.sync_copy(dst_hbm.at[pl.ds(base, C)], dst_v)
        pltpu.async_copy(h_hbm.at[src_v], rows_v, sem).wait()
        pltpu.sync_copy(rows_v, acc_sh.at[dst_v], add=True)
        return carry

    lax.fori_loop(0, chunks, chunk, 0)
    plsc.subcore_barrier()
    pltpu.sync_copy(acc_sh.at[pl.ds(sid * RPT, RPT)],
                    out_hbm.at[pl.ds(cid * NPAD + sid * RPT, RPT)])


_B = 1000
_NB = N // _B


def _tc_layer0_body(a0, a1, cnt, h, wlT, wrT, b, g, be, out):
    agg = a0[...] + a1[...]
    _tc_layer_tail(agg, cnt, h[...], wlT, wrT, b, g, be, out)


def _tc_layer12_body(a0, a1, cnt, h0, h1, wlT, wrT, b, g, be, out):
    agg = jnp.concatenate([a0[...], a1[...]], axis=-1)
    h = jnp.concatenate([h0[...], h1[...]], axis=-1)
    _tc_layer_tail(agg, cnt, h, wlT, wrT, b, g, be, out)


def _tc_layer_tail(agg, cnt, h, wlT, wrT, b, g, be, out):
    inv = 1.0 / jnp.maximum(cnt[...], 1.0)
    m = (jnp.dot(agg * inv, wlT[...], preferred_element_type=jnp.float32)
         + jnp.dot(h, wrT[...], preferred_element_type=jnp.float32)
         + b[...])
    mu = jnp.mean(m, axis=-1, keepdims=True)
    xc = m - mu
    var = jnp.mean(xc * xc, axis=-1, keepdims=True)
    y = xc * lax.rsqrt(var + 1e-5) * g[...] + be[...]
    y = jnp.maximum(y, 0.0)
    out[0] = y[:, : H // 2]
    out[1] = y[:, H // 2:]


def _row_spec(w):
    return pl.BlockSpec((_B, w), lambda i: (i, 0))


def _full_spec(r, c):
    return pl.BlockSpec((r, c), lambda i: (0, 0))


def _tc_layer0(a0, a1, cnt, x, wlT, wrT, b, g, be):
    return pl.pallas_call(
        _tc_layer0_body,
        grid=(_NB,),
        in_specs=[
            _row_spec(D), _row_spec(D), _row_spec(1), _row_spec(D),
            _full_spec(D, H), _full_spec(D, H),
            _full_spec(1, H), _full_spec(1, H), _full_spec(1, H),
        ],
        out_specs=pl.BlockSpec((2, _B, H // 2), lambda i: (0, i, 0)),
        out_shape=jax.ShapeDtypeStruct((2, N, H // 2), jnp.float32),
    )(a0, a1, cnt, x, wlT, wrT, b, g, be)


def _tc_layer12(a0, a1, cnt, h0, h1, wlT, wrT, b, g, be):
    return pl.pallas_call(
        _tc_layer12_body,
        grid=(_NB,),
        in_specs=[
            _row_spec(H // 2), _row_spec(H // 2), _row_spec(1),
            _row_spec(H // 2), _row_spec(H // 2),
            _full_spec(H, H), _full_spec(H, H),
            _full_spec(1, H), _full_spec(1, H), _full_spec(1, H),
        ],
        out_specs=pl.BlockSpec((2, _B, H // 2), lambda i: (0, i, 0)),
        out_shape=jax.ShapeDtypeStruct((2, N, H // 2), jnp.float32),
    )(a0, a1, cnt, h0, h1, wlT, wrT, b, g, be)


def _tc_pool_body(h0, h1, batch, fcWT, fcb, out, accs, accc):
    i = pl.program_id(0)

    @pl.when(i == 0)
    def _():
        accs[...] = jnp.zeros_like(accs)
        accc[...] = jnp.zeros_like(accc)

    h = jnp.concatenate([h0[...], h1[...]], axis=-1)
    gids = lax.broadcasted_iota(jnp.int32, (1, G), 1)
    onehotT = (batch[...] == gids).astype(jnp.float32)
    accs[...] += lax.dot_general(onehotT, h, (((0,), (0,)), ((), ())),
                                 preferred_element_type=jnp.float32)
    ones = jnp.ones((_B, 128), jnp.float32)
    accc[...] += lax.dot_general(onehotT, ones, (((0,), (0,)), ((), ())),
                                 preferred_element_type=jnp.float32)

    @pl.when(i == _NB - 1)
    def _():
        pooled = accs[...] / jnp.maximum(accc[:, :1], 1.0)
        z = jnp.dot(pooled, fcWT[...], preferred_element_type=jnp.float32)
        out[...] = jnp.tanh(z + fcb[...])


def _tc_pool(h0, h1, batch2d, fcWT, fcb):
    return pl.pallas_call(
        _tc_pool_body,
        grid=(_NB,),
        in_specs=[
            _row_spec(H // 2), _row_spec(H // 2), _row_spec(1),
            _full_spec(H, H), _full_spec(1, H),
        ],
        out_specs=pl.BlockSpec((G, H), lambda i: (0, 0)),
        out_shape=jax.ShapeDtypeStruct((G, H), jnp.float32),
        scratch_shapes=[
            pltpu.VMEM((G, H), jnp.float32),
            pltpu.VMEM((G, 128), jnp.float32),
        ],
    )(h0, h1, batch2d, fcWT, fcb)


def kernel(x, edge_index, batch, Wl0, Wr0, b0, Wl1, Wr1, b1,
           Wl2, Wr2, b2, gamma, beta, fcW, fcb):
    src, dst = edge_index[0], edge_index[1]
    srcs = jnp.concatenate([src, src + N])
    zeros_pad = jnp.zeros((NPAD, 128), jnp.float32)
    ones_c128 = jnp.ones((C, 128), jnp.float32)
    batch2d = batch.reshape(N, 1)

    cntp = _sc_count(dst, ones_c128, zeros_pad)
    cnt = cntp[:N, :1] + cntp[NPAD:NPAD + N, :1]

    b0r = b0.reshape(1, H)
    b1r = b1.reshape(1, H)
    b2r = b2.reshape(1, H)
    gr = gamma.reshape(1, H)
    ber = beta.reshape(1, H)
    fcbr = fcb.reshape(1, H)

    aggp0 = _sc_agg_l0(x, src, dst, zeros_pad)
    h1f = _tc_layer0(aggp0[:N], aggp0[NPAD:NPAD + N], cnt,
                     x, Wl0.T, Wr0.T, b0r, gr, ber)

    aggp1 = _sc_agg(h1f.reshape(2 * N, H // 2), srcs, dst, zeros_pad)
    h2f = _tc_layer12(aggp1[:N], aggp1[NPAD:NPAD + N], cnt,
                      h1f[0], h1f[1], Wl1.T, Wr1.T, b1r, gr, ber)

    aggp2 = _sc_agg(h2f.reshape(2 * N, H // 2), srcs, dst, zeros_pad)
    h3f = _tc_layer12(aggp2[:N], aggp2[NPAD:NPAD + N], cnt,
                      h2f[0], h2f[1], Wl2.T, Wr2.T, b2r, gr, ber)

    return _tc_pool(h3f[0], h3f[1], batch2d, fcW.T, fcbr)


# speedup vs baseline: 7.3189x; 2.0998x over previous
import functools

import jax
import jax.numpy as jnp
from jax import lax
from jax.experimental import pallas as pl
from jax.experimental.pallas import tpu as pltpu
from jax.experimental.pallas import tpu_sc as plsc

N = 10000
E = 320000
D = 128
H = 256
G = 64

NUM_TILES = 16
C = 125
NCHUNK = E // C
NPAD = 10240
RPT = NPAD // NUM_TILES
W = 128

_mesh = plsc.VectorSubcoreMesh(core_axis_name="c", subcore_axis_name="s")


@functools.partial(
    pl.kernel,
    mesh=_mesh,
    out_type=jax.ShapeDtypeStruct((2 * NPAD, W), jnp.float32),
    scratch_types=[
        pltpu.VMEM((NCHUNK // 32, C), jnp.int32),
        pltpu.VMEM((C, W), jnp.float32),
        pltpu.VMEM_SHARED((NPAD, W), jnp.float32),
        pltpu.SemaphoreType.DMA,
    ],
)
def _sc_count(dst2d_hbm, ones_hbm, zeros_hbm, out_hbm,
              dst2d_v, ones_v, acc_sh, sem):
    cid = lax.axis_index("c")
    sid = lax.axis_index("s")
    chunks = NCHUNK // 32
    base = cid * (NCHUNK // 2) + sid * chunks

    pltpu.sync_copy(zeros_hbm.at[pl.ds(sid * RPT, RPT)],
                    acc_sh.at[pl.ds(sid * RPT, RPT)])
    pltpu.sync_copy(dst2d_hbm.at[pl.ds(base, chunks)], dst2d_v)
    pltpu.sync_copy(ones_hbm, ones_v)
    plsc.subcore_barrier()

    def chunk(k, carry):
        pltpu.async_copy(ones_v, acc_sh.at[dst2d_v.at[k]], sem, add=True)

        @pl.when(k > 0)
        def _():
            pltpu.make_async_copy(ones_v, acc_sh.at[dst2d_v.at[0]], sem).wait()

        return carry

    lax.fori_loop(0, chunks, chunk, 0)
    pltpu.make_async_copy(ones_v, acc_sh.at[dst2d_v.at[0]], sem).wait()
    plsc.subcore_barrier()
    pltpu.sync_copy(acc_sh.at[pl.ds(sid * RPT, RPT)],
                    out_hbm.at[pl.ds(cid * NPAD + sid * RPT, RPT)])


GB = 8


def _make_sc_agg(batches, base_fn):
    @functools.partial(
        pl.kernel,
        mesh=_mesh,
        out_type=jax.ShapeDtypeStruct((2 * NPAD, W), jnp.float32),
        scratch_types=[
            pltpu.VMEM((2, GB, C), jnp.int32),
            pltpu.VMEM((2, GB, C), jnp.int32),
            pltpu.VMEM((C, W), jnp.float32),
            pltpu.VMEM((C, W), jnp.float32),
            pltpu.VMEM_SHARED((NPAD, W), jnp.float32),
            pltpu.SemaphoreType.DMA,
            pltpu.SemaphoreType.DMA,
            pltpu.SemaphoreType.DMA,
            pltpu.SemaphoreType.DMA,
            pltpu.SemaphoreType.DMA,
            pltpu.SemaphoreType.DMA,
        ],
    )
    def sc_agg(h_hbm, comb_hbm, zeros_hbm, out_hbm,
               ib0, ib1, rows0_v, rows1_v, acc_sh,
               semi0, semi1, semg0, semg1, sems0, sems1):
        cid = lax.axis_index("c")
        sid = lax.axis_index("s")
        bbase = base_fn(cid, sid)
        pltpu.sync_copy(zeros_hbm.at[pl.ds(sid * RPT, RPT)],
                        acc_sh.at[pl.ds(sid * RPT, RPT)])
        plsc.subcore_barrier()

        pltpu.async_copy(comb_hbm.at[bbase], ib0, semi0)
        pltpu.async_copy(comb_hbm.at[bbase + 1], ib1, semi1)

        def batch_body(m, ib, semi):
            pltpu.make_async_copy(comb_hbm.at[bbase + m], ib, semi).wait()
            g_prev = pltpu.async_copy(h_hbm.at[ib.at[0, 0]], rows0_v, semg0)
            for j in range(GB // 2):
                a, b = 2 * j, 2 * j + 1
                g_prev.wait()
                gb = pltpu.async_copy(h_hbm.at[ib.at[0, b]], rows1_v, semg1)
                sa = pltpu.async_copy(rows0_v, acc_sh.at[ib.at[1, a]],
                                      sems0, add=True)
                gb.wait()
                sa.wait()
                if j < GB // 2 - 1:
                    g_prev = pltpu.async_copy(h_hbm.at[ib.at[0, b + 1]],
                                              rows0_v, semg0)
                pltpu.async_copy(rows1_v, acc_sh.at[ib.at[1, b]],
                                 sems1, add=True).wait()

            @pl.when(m + 2 < batches)
            def _():
                pltpu.async_copy(comb_hbm.at[bbase + m + 2], ib, semi)

        def outer(t, carry):
            batch_body(2 * t, ib0, semi0)
            batch_body(2 * t + 1, ib1, semi1)
            return carry

        lax.fori_loop(0, batches // 2, outer, 0)
        plsc.subcore_barrier()
        pltpu.sync_copy(acc_sh.at[pl.ds(sid * RPT, RPT)],
                        out_hbm.at[pl.ds(cid * NPAD + sid * RPT, RPT)])

    return sc_agg


_NB0 = NCHUNK // 32 // GB
_NB12 = NCHUNK // 16 // GB

_sc_agg_l0 = _make_sc_agg(
    _NB0, lambda cid, sid: cid * (16 * _NB0) + sid * _NB0)
_sc_agg_12 = _make_sc_agg(
    _NB12, lambda cid, sid: cid * (16 * _NB12) + sid * _NB12)


_B = 1000
_NB = N // _B


def _tc_layer0_body(a0, a1, cnt, h, wlT, wrT, b, g, be, out):
    agg = a0[...] + a1[...]
    _tc_layer_tail(agg, cnt, h[...], wlT, wrT, b, g, be, out)


def _tc_layer12_body(a0, a1, cnt, h0, h1, wlT, wrT, b, g, be, out):
    agg = jnp.concatenate([a0[...], a1[...]], axis=-1)
    h = jnp.concatenate([h0[...], h1[...]], axis=-1)
    _tc_layer_tail(agg, cnt, h, wlT, wrT, b, g, be, out)


def _tc_layer_tail(agg, cnt, h, wlT, wrT, b, g, be, out):
    inv = 1.0 / jnp.maximum(cnt[...], 1.0)
    m = (jnp.dot(agg * inv, wlT[...], preferred_element_type=jnp.float32)
         + jnp.dot(h, wrT[...], preferred_element_type=jnp.float32)
         + b[...])
    mu = jnp.mean(m, axis=-1, keepdims=True)
    xc = m - mu
    var = jnp.mean(xc * xc, axis=-1, keepdims=True)
    y = xc * lax.rsqrt(var + 1e-5) * g[...] + be[...]
    y = jnp.maximum(y, 0.0)
    out[0] = y[:, : H // 2]
    out[1] = y[:, H // 2:]


def _row_spec(w):
    return pl.BlockSpec((_B, w), lambda i: (i, 0))


def _full_spec(r, c):
    return pl.BlockSpec((r, c), lambda i: (0, 0))


def _tc_layer0(a0, a1, cnt, x, wlT, wrT, b, g, be):
    return pl.pallas_call(
        _tc_layer0_body,
        grid=(_NB,),
        in_specs=[
            _row_spec(D), _row_spec(D), _row_spec(1), _row_spec(D),
            _full_spec(D, H), _full_spec(D, H),
            _full_spec(1, H), _full_spec(1, H), _full_spec(1, H),
        ],
        out_specs=pl.BlockSpec((2, _B, H // 2), lambda i: (0, i, 0)),
        out_shape=jax.ShapeDtypeStruct((2, N, H // 2), jnp.float32),
    )(a0, a1, cnt, x, wlT, wrT, b, g, be)


def _tc_layer12(a0, a1, cnt, h0, h1, wlT, wrT, b, g, be):
    return pl.pallas_call(
        _tc_layer12_body,
        grid=(_NB,),
        in_specs=[
            _row_spec(H // 2), _row_spec(H // 2), _row_spec(1),
            _row_spec(H // 2), _row_spec(H // 2),
            _full_spec(H, H), _full_spec(H, H),
            _full_spec(1, H), _full_spec(1, H), _full_spec(1, H),
        ],
        out_specs=pl.BlockSpec((2, _B, H // 2), lambda i: (0, i, 0)),
        out_shape=jax.ShapeDtypeStruct((2, N, H // 2), jnp.float32),
    )(a0, a1, cnt, h0, h1, wlT, wrT, b, g, be)


def _tc_pool_body(h0, h1, batch, fcWT, fcb, out, accs, accc):
    i = pl.program_id(0)

    @pl.when(i == 0)
    def _():
        accs[...] = jnp.zeros_like(accs)
        accc[...] = jnp.zeros_like(accc)

    h = jnp.concatenate([h0[...], h1[...]], axis=-1)
    gids = lax.broadcasted_iota(jnp.int32, (1, G), 1)
    onehotT = (batch[...] == gids).astype(jnp.float32)
    accs[...] += lax.dot_general(onehotT, h, (((0,), (0,)), ((), ())),
                                 preferred_element_type=jnp.float32)
    ones = jnp.ones((_B, 128), jnp.float32)
    accc[...] += lax.dot_general(onehotT, ones, (((0,), (0,)), ((), ())),
                                 preferred_element_type=jnp.float32)

    @pl.when(i == _NB - 1)
    def _():
        pooled = accs[...] / jnp.maximum(accc[:, :1], 1.0)
        z = jnp.dot(pooled, fcWT[...], preferred_element_type=jnp.float32)
        out[...] = jnp.tanh(z + fcb[...])


def _tc_pool(h0, h1, batch2d, fcWT, fcb):
    return pl.pallas_call(
        _tc_pool_body,
        grid=(_NB,),
        in_specs=[
            _row_spec(H // 2), _row_spec(H // 2), _row_spec(1),
            _full_spec(H, H), _full_spec(1, H),
        ],
        out_specs=pl.BlockSpec((G, H), lambda i: (0, 0)),
        out_shape=jax.ShapeDtypeStruct((G, H), jnp.float32),
        scratch_shapes=[
            pltpu.VMEM((G, H), jnp.float32),
            pltpu.VMEM((G, 128), jnp.float32),
        ],
    )(h0, h1, batch2d, fcWT, fcb)


def kernel(x, edge_index, batch, Wl0, Wr0, b0, Wl1, Wr1, b1,
           Wl2, Wr2, b2, gamma, beta, fcW, fcb):
    src, dst = edge_index[0], edge_index[1]
    dst2d = dst.reshape(NCHUNK, C)
    srcA = src.reshape(NCHUNK // GB, GB, C)
    dstA = dst.reshape(NCHUNK // GB, GB, C)
    comb0 = jnp.stack([srcA, dstA], axis=1)
    comb12 = jnp.concatenate(
        [comb0, jnp.stack([srcA + N, dstA], axis=1)], axis=0)
    zeros_pad = jnp.zeros((NPAD, W), jnp.float32)
    ones_cw = jnp.ones((C, W), jnp.float32)
    batch2d = batch.reshape(N, 1)

    cntp = _sc_count(dst2d, ones_cw, zeros_pad)
    cnt = cntp[:N, :1] + cntp[NPAD:NPAD + N, :1]

    b0r = b0.reshape(1, H)
    b1r = b1.reshape(1, H)
    b2r = b2.reshape(1, H)
    gr = gamma.reshape(1, H)
    ber = beta.reshape(1, H)
    fcbr = fcb.reshape(1, H)

    aggp0 = _sc_agg_l0(x, comb0, zeros_pad)
    h1f = _tc_layer0(aggp0[:N], aggp0[NPAD:NPAD + N], cnt,
                     x, Wl0.T, Wr0.T, b0r, gr, ber)

    aggp1 = _sc_agg_12(h1f.reshape(2 * N, H // 2), comb12, zeros_pad)
    h2f = _tc_layer12(aggp1[:N], aggp1[NPAD:NPAD + N], cnt,
                      h1f[0], h1f[1], Wl1.T, Wr1.T, b1r, gr, ber)

    aggp2 = _sc_agg_12(h2f.reshape(2 * N, H // 2), comb12, zeros_pad)
    h3f = _tc_layer12(aggp2[:N], aggp2[NPAD:NPAD + N], cnt,
                      h2f[0], h2f[1], Wl2.T, Wr2.T, b2r, gr, ber)

    return _tc_pool(h3f[0], h3f[1], batch2d, fcW.T, fcbr)


# speedup vs baseline: 7.8657x; 1.0747x over previous
import functools

import jax
import jax.numpy as jnp
from jax import lax
from jax.experimental import pallas as pl
from jax.experimental.pallas import tpu as pltpu
from jax.experimental.pallas import tpu_sc as plsc

N = 10000
E = 320000
D = 128
H = 256
G = 64

NUM_TILES = 16
C = 125
NCHUNK = E // C
GB = 8
NPAD = 10240
RPT = NPAD // NUM_TILES
W = 128

_mesh = plsc.VectorSubcoreMesh(core_axis_name="c", subcore_axis_name="s")


def _acc_out(shape=(NPAD, W)):
    return [jax.ShapeDtypeStruct(shape, jnp.float32),
            jax.ShapeDtypeStruct(shape, jnp.float32)]


@functools.partial(
    pl.kernel,
    mesh=_mesh,
    out_type=_acc_out(),
    scratch_types=[
        pltpu.VMEM((NCHUNK // 32, C), jnp.int32),
        pltpu.VMEM((C, W), jnp.float32),
        pltpu.VMEM_SHARED((NPAD, W), jnp.float32),
        pltpu.SemaphoreType.DMA,
    ],
)
def _sc_count(dst2d_hbm, ones_hbm, zeros_hbm, out0_hbm, out1_hbm,
              dst2d_v, ones_v, acc_sh, sem):
    cid = lax.axis_index("c")
    sid = lax.axis_index("s")
    chunks = NCHUNK // 32
    base = cid * (NCHUNK // 2) + sid * chunks

    pltpu.sync_copy(zeros_hbm.at[pl.ds(sid * RPT, RPT)],
                    acc_sh.at[pl.ds(sid * RPT, RPT)])
    pltpu.sync_copy(dst2d_hbm.at[pl.ds(base, chunks)], dst2d_v)
    pltpu.sync_copy(ones_hbm, ones_v)
    plsc.subcore_barrier()

    def chunk(k, carry):
        pltpu.async_copy(ones_v, acc_sh.at[dst2d_v.at[k]], sem, add=True)

        @pl.when(k > 0)
        def _():
            pltpu.make_async_copy(ones_v, acc_sh.at[dst2d_v.at[0]], sem).wait()

        return carry

    lax.fori_loop(0, chunks, chunk, 0)
    pltpu.make_async_copy(ones_v, acc_sh.at[dst2d_v.at[0]], sem).wait()
    plsc.subcore_barrier()

    @pl.when(cid == 0)
    def _():
        pltpu.sync_copy(acc_sh.at[pl.ds(sid * RPT, RPT)],
                        out0_hbm.at[pl.ds(sid * RPT, RPT)])

    @pl.when(cid == 1)
    def _():
        pltpu.sync_copy(acc_sh.at[pl.ds(sid * RPT, RPT)],
                        out1_hbm.at[pl.ds(sid * RPT, RPT)])


def _make_sc_agg(batches, base_fn):
    @functools.partial(
        pl.kernel,
        mesh=_mesh,
        out_type=_acc_out(),
        scratch_types=[
            pltpu.VMEM((2, GB, C), jnp.int32),
            pltpu.VMEM((2, GB, C), jnp.int32),
            pltpu.VMEM((C, W), jnp.float32),
            pltpu.VMEM((C, W), jnp.float32),
            pltpu.VMEM_SHARED((NPAD, W), jnp.float32),
            pltpu.SemaphoreType.DMA,
            pltpu.SemaphoreType.DMA,
            pltpu.SemaphoreType.DMA,
            pltpu.SemaphoreType.DMA,
            pltpu.SemaphoreType.DMA,
            pltpu.SemaphoreType.DMA,
        ],
    )
    def sc_agg(h_hbm, comb_hbm, zeros_hbm, out0_hbm, out1_hbm,
               ib0, ib1, rows0_v, rows1_v, acc_sh,
               semi0, semi1, semg0, semg1, sems0, sems1):
        cid = lax.axis_index("c")
        sid = lax.axis_index("s")
        bbase = base_fn(cid, sid)
        rows = (rows0_v, rows1_v)
        semg = (semg0, semg1)
        sems = (sems0, sems1)

        pltpu.sync_copy(zeros_hbm.at[pl.ds(sid * RPT, RPT)],
                        acc_sh.at[pl.ds(sid * RPT, RPT)])
        plsc.subcore_barrier()

        pltpu.async_copy(comb_hbm.at[bbase], ib0, semi0)
        pltpu.async_copy(comb_hbm.at[bbase + 1], ib1, semi1)
        pltpu.make_async_copy(comb_hbm.at[bbase], ib0, semi0).wait()
        pltpu.async_copy(h_hbm.at[ib0.at[0, 0]], rows0_v, semg0)

        def batch_body(m, ib, ib_next, semi_self, semi_next):
            s_prev = None
            for k in range(GB):
                p = k % 2
                if k == 0:
                    pltpu.make_async_copy(h_hbm.at[ib.at[0, 0]], rows0_v,
                                          semg0).wait()
                else:
                    g_cur.wait()
                if s_prev is not None:
                    s_prev.wait()
                if k + 1 < GB:
                    g_cur = pltpu.async_copy(h_hbm.at[ib.at[0, k + 1]],
                                             rows[(k + 1) % 2],
                                             semg[(k + 1) % 2])
                else:
                    @pl.when(m + 1 < batches)
                    def _():
                        pltpu.make_async_copy(comb_hbm.at[bbase + m + 1],
                                              ib_next, semi_next).wait()
                        pltpu.async_copy(h_hbm.at[ib_next.at[0, 0]],
                                         rows0_v, semg0)
                s_prev = pltpu.async_copy(rows[p], acc_sh.at[ib.at[1, k]],
                                          sems[p], add=True)
            s_prev.wait()

            @pl.when(m + 2 < batches)
            def _():
                pltpu.async_copy(comb_hbm.at[bbase + m + 2], ib, semi_self)

        def outer(t, carry):
            batch_body(2 * t, ib0, ib1, semi0, semi1)
            batch_body(2 * t + 1, ib1, ib0, semi1, semi0)
            return carry

        lax.fori_loop(0, batches // 2, outer, 0)
        plsc.subcore_barrier()

        @pl.when(cid == 0)
        def _():
            pltpu.sync_copy(acc_sh.at[pl.ds(sid * RPT, RPT)],
                            out0_hbm.at[pl.ds(sid * RPT, RPT)])

        @pl.when(cid == 1)
        def _():
            pltpu.sync_copy(acc_sh.at[pl.ds(sid * RPT, RPT)],
                            out1_hbm.at[pl.ds(sid * RPT, RPT)])

    return sc_agg


assert GB % 2 == 0

_NB0 = NCHUNK // 32 // GB
_NB12 = NCHUNK // 16 // GB

_sc_agg_l0 = _make_sc_agg(
    _NB0, lambda cid, sid: cid * (16 * _NB0) + sid * _NB0)
_sc_agg_12 = _make_sc_agg(
    _NB12, lambda cid, sid: cid * (16 * _NB12) + sid * _NB12)


_B = 1000
_NB = N // _B


def _cnt_inv(c0, c1):
    return 1.0 / jnp.maximum(c0[...] + c1[...], 1.0)


def _tc_layer0_body(a0, a1, c0, c1, h, wlT, wrT, b, g, be, out):
    agg = a0[...] + a1[...]
    _tc_layer_tail(agg, _cnt_inv(c0, c1), h[...], wlT, wrT, b, g, be, out)


def _tc_layer12_body(a0, a1, c0, c1, h3d, wlT, wrT, b, g, be, out):
    agg = jnp.concatenate([a0[...], a1[...]], axis=-1)
    h = jnp.concatenate([h3d[0], h3d[1]], axis=-1)
    _tc_layer_tail(agg, _cnt_inv(c0, c1), h, wlT, wrT, b, g, be, out)


def _tc_layer_tail(agg, inv, h, wlT, wrT, b, g, be, out):
    m = (jnp.dot(agg * inv, wlT[...], preferred_element_type=jnp.float32)
         + jnp.dot(h, wrT[...], preferred_element_type=jnp.float32)
         + b[...])
    mu = jnp.mean(m, axis=-1, keepdims=True)
    xc = m - mu
    var = jnp.mean(xc * xc, axis=-1, keepdims=True)
    y = xc * lax.rsqrt(var + 1e-5) * g[...] + be[...]
    y = jnp.maximum(y, 0.0)
    out[0] = y[:, : H // 2]
    out[1] = y[:, H // 2:]


def _row_spec(w):
    return pl.BlockSpec((_B, w), lambda i: (i, 0))


def _full_spec(r, c):
    return pl.BlockSpec((r, c), lambda i: (0, 0))


_cnt_spec = pl.BlockSpec((_B, 1), lambda i: (i, 0))
_h3d_spec = pl.BlockSpec((2, _B, H // 2), lambda i: (0, i, 0))


def _tc_layer0(a0, a1, c0, c1, x, wlT, wrT, b, g, be):
    return pl.pallas_call(
        _tc_layer0_body,
        grid=(_NB,),
        in_specs=[
            _row_spec(D), _row_spec(D), _cnt_spec, _cnt_spec, _row_spec(D),
            _full_spec(D, H), _full_spec(D, H),
            _full_spec(1, H), _full_spec(1, H), _full_spec(1, H),
        ],
        out_specs=_h3d_spec,
        out_shape=jax.ShapeDtypeStruct((2, N, H // 2), jnp.float32),
    )(a0, a1, c0, c1, x, wlT, wrT, b, g, be)


def _tc_layer12(a0, a1, c0, c1, h3d, wlT, wrT, b, g, be):
    return pl.pallas_call(
        _tc_layer12_body,
        grid=(_NB,),
        in_specs=[
            _row_spec(H // 2), _row_spec(H // 2), _cnt_spec, _cnt_spec,
            _h3d_spec,
            _full_spec(H, H), _full_spec(H, H),
            _full_spec(1, H), _full_spec(1, H), _full_spec(1, H),
        ],
        out_specs=_h3d_spec,
        out_shape=jax.ShapeDtypeStruct((2, N, H // 2), jnp.float32),
    )(a0, a1, c0, c1, h3d, wlT, wrT, b, g, be)


def _tc_pool_body(h3d, batch, fcWT, fcb, out, accs, accc):
    i = pl.program_id(0)

    @pl.when(i == 0)
    def _():
        accs[...] = jnp.zeros_like(accs)
        accc[...] = jnp.zeros_like(accc)

    h = jnp.concatenate([h3d[0], h3d[1]], axis=-1)
    gids = lax.broadcasted_iota(jnp.int32, (1, G), 1)
    onehotT = (batch[...] == gids).astype(jnp.float32)
    accs[...] += lax.dot_general(onehotT, h, (((0,), (0,)), ((), ())),
                                 preferred_element_type=jnp.float32)
    ones = jnp.ones((_B, 128), jnp.float32)
    accc[...] += lax.dot_general(onehotT, ones, (((0,), (0,)), ((), ())),
                                 preferred_element_type=jnp.float32)

    @pl.when(i == _NB - 1)
    def _():
        pooled = accs[...] / jnp.maximum(accc[:, :1], 1.0)
        z = jnp.dot(pooled, fcWT[...], preferred_element_type=jnp.float32)
        out[...] = jnp.tanh(z + fcb[...])


def _tc_pool(h3d, batch2d, fcWT, fcb):
    return pl.pallas_call(
        _tc_pool_body,
        grid=(_NB,),
        in_specs=[
            _h3d_spec, _cnt_spec,
            _full_spec(H, H), _full_spec(1, H),
        ],
        out_specs=pl.BlockSpec((G, H), lambda i: (0, 0)),
        out_shape=jax.ShapeDtypeStruct((G, H), jnp.float32),
        scratch_shapes=[
            pltpu.VMEM((G, H), jnp.float32),
            pltpu.VMEM((G, 128), jnp.float32),
        ],
    )(h3d, batch2d, fcWT, fcb)


def kernel(x, edge_index, batch, Wl0, Wr0, b0, Wl1, Wr1, b1,
           Wl2, Wr2, b2, gamma, beta, fcW, fcb):
    src, dst = edge_index[0], edge_index[1]
    dst2d = dst.reshape(NCHUNK, C)
    srcA = src.reshape(NCHUNK // GB, GB, C)
    dstA = dst.reshape(NCHUNK // GB, GB, C)
    comb0 = jnp.stack([srcA, dstA], axis=1)
    comb12 = jnp.concatenate(
        [comb0, jnp.stack([srcA + N, dstA], axis=1)], axis=0)
    zeros_pad = jnp.zeros((NPAD, W), jnp.float32)
    ones_cw = jnp.ones((C, W), jnp.float32)
    batch2d = batch.reshape(N, 1)

    cnt0, cnt1 = _sc_count(dst2d, ones_cw, zeros_pad)
    c0 = cnt0[:, :1]
    c1 = cnt1[:, :1]

    b0r = b0.reshape(1, H)
    b1r = b1.reshape(1, H)
    b2r = b2.reshape(1, H)
    gr = gamma.reshape(1, H)
    ber = beta.reshape(1, H)
    fcbr = fcb.reshape(1, H)

    a00, a01 = _sc_agg_l0(x, comb0, zeros_pad)
    h1f = _tc_layer0(a00, a01, c0, c1, x, Wl0.T, Wr0.T, b0r, gr, ber)

    a10, a11 = _sc_agg_12(h1f.reshape(2 * N, H // 2), comb12, zeros_pad)
    h2f = _tc_layer12(a10, a11, c0, c1, h1f, Wl1.T, Wr1.T, b1r, gr, ber)

    a20, a21 = _sc_agg_12(h2f.reshape(2 * N, H // 2), comb12, zeros_pad)
    h3f = _tc_layer12(a20, a21, c0, c1, h2f, Wl2.T, Wr2.T, b2r, gr, ber)

    return _tc_pool(h3f, batch2d, fcW.T, fcbr)
